# Initial kernel scaffold; baseline (speedup 1.0000x reference)
import jax
import jax.numpy as jnp
from jax.experimental import pallas as pl


def kernel(x, idxn, segment_ids):
    raise NotImplementedError("write your pallas kernel here")


# speedup vs baseline: 6.6765x; 6.6765x over previous
import dataclasses
import functools

import jax
import jax.numpy as jnp
from jax import lax
from jax.experimental import pallas as pl
from jax.experimental.pallas import tpu as pltpu
from jax.experimental.pallas import tpu_sc as plsc

N_NODES = 10000
D = 128
E = 320000
N_OUT = 10000

NC = 2
NS = 16
NW = NC * NS
SPW = 320
PAD_OUT = NW * SPW
CH = 128
TRASH = SPW
ACC_ROWS = SPW + 8
RB = 64


def _pool_kernel(x_hbm, idx_hbm, seg_hbm, bnd_hbm, out_hbm,
                 acc_sh, work, cnt, zrow, rows,
                 idx_v, seg_v, loc_v, bnd_v):
    sid = lax.axis_index("subcore")
    wid = lax.axis_index("core") * NS + sid
    seg_lo = wid * SPW
    slab = sid * ACC_ROWS

    zeros16 = jnp.zeros((16,), jnp.float32)
    ones16 = jnp.full((16,), 1.0, jnp.float32)
    lane = lax.iota(jnp.int32, 16)

    @pl.loop(0, 8)
    def _(i):
        for j in range(D // 16):
            zrow[i, pl.ds(j * 16, 16)] = zeros16

    @pl.loop(0, ACC_ROWS // 8)
    def _(i):
        pltpu.sync_copy(zrow, acc_sh.at[pl.ds(slab + i * 8, 8)])

    @pl.loop(0, ACC_ROWS)
    def _(i):
        cnt[pl.ds(i * 16, 16)] = zeros16

    pltpu.sync_copy(bnd_hbm, bnd_v)
    widv = jnp.full((16,), wid, jnp.int32)
    e_lo = jnp.max(plsc.load_gather(bnd_v, [widv]))
    e_hi = jnp.max(plsc.load_gather(bnd_v, [widv + 1]))
    c_lo = e_lo // CH
    c_hi = (e_hi + CH - 1) // CH

    def chunk_body(c, carry):
        base = pl.multiple_of(c * CH, CH)
        pltpu.sync_copy(idx_hbm.at[pl.ds(base, CH)], idx_v)
        pltpu.sync_copy(seg_hbm.at[pl.ds(base, CH)], seg_v)
        for j in range(CH // 16):
            s = seg_v[pl.ds(j * 16, 16)]
            sl = s - seg_lo
            ok = (sl >= 0) & (sl < SPW)
            sl = jnp.where(ok, sl, TRASH)
            loc_v[pl.ds(j * 16, 16)] = slab + sl
            plsc.addupdate_scatter(cnt, [sl * 16 + lane], ones16)
        pltpu.sync_copy(x_hbm.at[idx_v], rows)
        pltpu.sync_copy(rows, acc_sh.at[loc_v], add=True)
        return carry

    lax.fori_loop(c_lo, c_hi, chunk_body, 0)

    @pl.loop(0, SPW // RB)
    def _(t):
        pltpu.sync_copy(acc_sh.at[pl.ds(slab + t * RB, RB)], work)

        @pl.loop(0, RB)
        def _(i):
            cvec = cnt[pl.ds((t * RB + i) * 16, 16)]
            total = jnp.full((16,), jnp.sum(cvec), jnp.float32)
            recip = 1.0 / jnp.maximum(total, 1.0)
            for j in range(D // 16):
                work[i, pl.ds(j * 16, 16)] = work[i, pl.ds(j * 16, 16)] * recip

        pltpu.sync_copy(work, out_hbm.at[pl.ds(seg_lo + t * RB, RB)])


@jax.jit
def kernel(x, idxn, segment_ids):
    seg_bnds = jnp.minimum(
        jnp.arange(NW + 1, dtype=jnp.int32) * SPW, N_OUT)
    ebounds = jnp.searchsorted(segment_ids, seg_bnds, side="left").astype(jnp.int32)
    ebounds = jnp.concatenate(
        [ebounds, jnp.zeros((40 - (NW + 1),), jnp.int32)])

    mesh = plsc.VectorSubcoreMesh(
        core_axis_name="core", subcore_axis_name="subcore")
    cp = pltpu.CompilerParams()
    if "needs_layout_passes" in pltpu.CompilerParams.__dataclass_fields__:
        cp = dataclasses.replace(cp, needs_layout_passes=False)
    run = pl.kernel(
        _pool_kernel,
        compiler_params=cp,
        out_type=jax.ShapeDtypeStruct((PAD_OUT, D), jnp.float32),
        mesh=mesh,
        scratch_types=[
            pltpu.VMEM_SHARED((NS * ACC_ROWS, D), jnp.float32),
            pltpu.VMEM((RB, D), jnp.float32),
            pltpu.VMEM((ACC_ROWS * 16,), jnp.float32),
            pltpu.VMEM((8, D), jnp.float32),
            pltpu.VMEM((CH, D), jnp.float32),
            pltpu.VMEM((CH,), jnp.int32),
            pltpu.VMEM((CH,), jnp.int32),
            pltpu.VMEM((CH,), jnp.int32),
            pltpu.VMEM((40,), jnp.int32),
        ],
    )
    out_pad = run(x, idxn, segment_ids, ebounds)
    return out_pad[:N_OUT]
